# baseline (speedup 1.0000x reference)
import functools

import jax
import jax.numpy as jnp
from jax import lax
from jax.experimental import pallas as pl
from jax.experimental.pallas import tpu as pltpu
from jax.experimental.pallas import tpu_sc as plsc

D = 128
HD = 64
L = 20000
C = 10000
E = 600000
NIT = 4

NSUB = 16
MICRO = 128
MACRO = 4
EPT = 37888
EP = EPT * NSUB
NMAC = EPT // (MICRO * MACRO)
RPT = EPT // MICRO
NROW = EP // MICRO

CA = 10112
LA = 20096
CPT = CA // NSUB
LPT = LA // NSUB

BL = 1000


def _sc_phase(r0, srcN, tbl, dstN, acc, idxs, idxd, rows, gsem, ssem, isem,
              n, nmac):
    nchunk = n // MICRO
    pltpu.async_copy(srcN.at[r0], idxs.at[0], isem)
    pltpu.async_copy(dstN.at[r0], idxd.at[0], isem)

    def drain(sem, par):
        for j in range(nchunk):
            pltpu.make_async_copy(
                tbl.at[pl.ds(0, MICRO)],
                rows.at[par, pl.ds(j * MICRO, MICRO)], sem).wait()

    def body(m, carry):
        b = lax.rem(m, 2)
        p = lax.rem(m, 3)

        @pl.when(m >= 2)
        def _():
            drain(ssem, b)

        pltpu.make_async_copy(srcN.at[r0], idxs.at[p], isem).wait()
        pltpu.make_async_copy(dstN.at[r0], idxd.at[p], isem).wait()
        pltpu.async_copy(tbl.at[idxs.at[p]], rows.at[b], gsem)

        @pl.when(m + 1 < nmac)
        def _():
            pn = lax.rem(m + 1, 3)
            pltpu.async_copy(srcN.at[r0 + m + 1], idxs.at[pn], isem)
            pltpu.async_copy(dstN.at[r0 + m + 1], idxd.at[pn], isem)

        @pl.when(m >= 1)
        def _():
            pm = lax.rem(m + 2, 3)
            drain(gsem, 1 - b)
            pltpu.async_copy(rows.at[1 - b], acc.at[idxd.at[pm]], ssem,
                             add=True)

        return carry

    lax.fori_loop(0, nmac, body, 0)
    blast = (nmac - 1) % 2
    plast = (nmac - 1) % 3
    drain(gsem, blast)
    pltpu.async_copy(rows.at[blast], acc.at[idxd.at[plast]], ssem, add=True)
    drain(ssem, 0)
    drain(ssem, 1)


def _make_seg_sum(nacc, per_tile, macro):
    n = macro * MICRO
    nmac = EPT // n

    @functools.partial(
        pl.kernel,
        out_type=(
            jax.ShapeDtypeStruct((nacc, HD), jnp.float32),
            jax.ShapeDtypeStruct((nacc, HD), jnp.float32),
        ),
        mesh=plsc.VectorSubcoreMesh(core_axis_name="c", subcore_axis_name="s"),
        compiler_params=pltpu.CompilerParams(use_tc_tiling_on_sc=False),
        scratch_types=[
            pltpu.VMEM_SHARED((nacc, HD), jnp.float32),
            pltpu.VMEM((3, n), jnp.int32),
            pltpu.VMEM((3, n), jnp.int32),
            pltpu.VMEM((2, n, HD), jnp.float32),
            pltpu.SemaphoreType.DMA,
            pltpu.SemaphoreType.DMA,
            pltpu.SemaphoreType.DMA,
        ],
    )
    def seg_sum(tbl0, tbl1, srcN, dstN, zrows,
                out0, out1, acc, idxs, idxd, rows, gsem, ssem, isem):
        core = lax.axis_index("c")
        sub = lax.axis_index("s")
        pltpu.sync_copy(zrows.at[pl.ds(0, per_tile)],
                        acc.at[pl.ds(sub * per_tile, per_tile)])
        plsc.subcore_barrier()

        @pl.when(core == 0)
        def _():
            _sc_phase(sub * nmac, srcN, tbl0, dstN, acc, idxs, idxd, rows,
                      gsem, ssem, isem, n, nmac)

        @pl.when(core == 1)
        def _():
            _sc_phase(sub * nmac, srcN, tbl1, dstN, acc, idxs, idxd, rows,
                      gsem, ssem, isem, n, nmac)

        plsc.subcore_barrier()

        @pl.when(core == 0)
        def _():
            pltpu.sync_copy(acc.at[pl.ds(sub * per_tile, per_tile)],
                            out0.at[pl.ds(sub * per_tile, per_tile)])

        @pl.when(core == 1)
        def _():
            pltpu.sync_copy(acc.at[pl.ds(sub * per_tile, per_tile)],
                            out1.at[pl.ds(sub * per_tile, per_tile)])

    return seg_sum


_sc_l2c = _make_seg_sum(CA, CPT, 4)
_sc_c2l = _make_seg_sum(LA, LPT, 2)


HR = LA // 16


@functools.partial(
    pl.kernel,
    out_type=(
        jax.ShapeDtypeStruct((HR, 16), jnp.float32),
        jax.ShapeDtypeStruct((CA // 16, 16), jnp.float32),
    ),
    mesh=plsc.VectorSubcoreMesh(core_axis_name="c", subcore_axis_name="s"),
    compiler_params=pltpu.CompilerParams(use_tc_tiling_on_sc=False,
                                         needs_layout_passes=False),
    scratch_types=[
        pltpu.VMEM_SHARED((HR, 16), jnp.float32),
        pltpu.VMEM((3, MACRO * MICRO), jnp.int32),
        pltpu.VMEM((HR, 16), jnp.float32),
        pltpu.VMEM((HR,), jnp.int32),
        pltpu.SemaphoreType.DMA,
        pltpu.SemaphoreType.DMA,
    ],
)
def _sc_degrees(ldst, cdst, z16, ident, ldeg, cdeg,
                dacc, idxd, hist, identv, ssem, isem):
    core = lax.axis_index("c")
    sub = lax.axis_index("s")
    pltpu.sync_copy(z16, hist)
    pltpu.sync_copy(ident, identv)

    @pl.when(sub == 0)
    def _():
        pltpu.sync_copy(z16, dacc)

    plsc.subcore_barrier()
    ones = jnp.ones((16,), jnp.float32)

    def deg_phase(dstN):
        r0 = sub * NMAC
        pltpu.async_copy(dstN.at[r0], idxd.at[0], isem)

        def body(m, carry):
            p = lax.rem(m, 3)
            pltpu.make_async_copy(dstN.at[r0], idxd.at[p], isem).wait()

            @pl.when(m + 1 < NMAC)
            def _():
                pn = lax.rem(m + 1, 3)
                pltpu.async_copy(dstN.at[r0 + m + 1], idxd.at[pn], isem)

            for k in range(MACRO * MICRO // 16):
                iv = idxd[p, pl.ds(k * 16, 16)]
                row = lax.shift_right_logical(iv, 4)
                col = lax.bitwise_and(iv, 15)
                plsc.addupdate_scatter(hist, [row, col], ones)

            return carry

        lax.fori_loop(0, NMAC, body, 0)

    @pl.when(core == 0)
    def _():
        deg_phase(ldst)

    @pl.when(core == 1)
    def _():
        deg_phase(cdst)

    pltpu.async_copy(hist, dacc.at[identv], ssem, add=True)
    pltpu.make_async_copy(z16, hist, ssem).wait()
    plsc.subcore_barrier()

    @pl.when((core == 0) & (sub == 0))
    def _():
        pltpu.sync_copy(dacc, ldeg)

    @pl.when((core == 1) & (sub == 0))
    def _():
        pltpu.sync_copy(dacc.at[pl.ds(0, CA // 16)], cdeg)


def _dot(a, b):
    return jnp.dot(a, b, preferred_element_type=jnp.float32)


def _l_step_body(pin, agg_lo, agg_hi, deg,
                 W1, b1, W2, b2, lW1, lb1, lW2, lb2, Wua, Wub, Wuc, bl,
                 lnew_ref, mlo_ref, mhi_ref, pout_ref):
    a = jnp.concatenate([agg_lo[...], agg_hi[...]], axis=1)
    d = deg[...][:, 0:1]
    s = jnp.where(d > 0, lax.rsqrt(d), 1.0)
    lnew = pin[...] + _dot(a * s, Wub[...])
    lnew_ref[...] = lnew
    h = jnp.maximum(_dot(lnew, W1[...]) + b1[...], 0.0)
    y = _dot(h, W2[...]) + b2[...]
    mlv = y * s
    mlo_ref[...] = mlv[:, :HD]
    mhi_ref[...] = mlv[:, HD:]
    h2 = jnp.maximum(_dot(lnew, lW1[...]) + lb1[...], 0.0)
    y2 = _dot(h2, lW2[...]) + lb2[...]
    z = _dot(y2, Wuc[...])
    r = lax.broadcasted_iota(jnp.int32, (BL, BL), 0)
    c = lax.broadcasted_iota(jnp.int32, (BL, BL), 1)
    sw = (c == (r ^ 1)).astype(jnp.float32)
    pout_ref[...] = _dot(lnew, Wua[...]) + _dot(sw, z) + bl[...]


def _c_step_body(pin, agg_lo, agg_hi, deg,
                 W1, b1, W2, b2, Wca, Wcb, bc,
                 cnew_ref, mlo_ref, mhi_ref, pout_ref):
    a = jnp.concatenate([agg_lo[...], agg_hi[...]], axis=1)
    d = deg[...][:, 0:1]
    s = jnp.where(d > 0, lax.rsqrt(d), 1.0)
    cnew = pin[...] + _dot(a * s, Wcb[...])
    cnew_ref[...] = cnew
    h = jnp.maximum(_dot(cnew, W1[...]) + b1[...], 0.0)
    y = _dot(h, W2[...]) + b2[...]
    mcv = y * s
    mlo_ref[...] = mcv[:, :HD]
    mhi_ref[...] = mcv[:, HD:]
    pout_ref[...] = _dot(cnew, Wca[...]) + bc[...]


def _row_spec(cols):
    return pl.BlockSpec((BL, cols), lambda i: (i, 0))


def _full_spec(shape):
    n = len(shape)
    return pl.BlockSpec(shape, lambda i: (0,) * n)


def _make_step(nrows, body, agg_widths, out_widths, wshapes):
    in_specs = ([_row_spec(D)] + [_row_spec(w) for w in agg_widths]
                + [_row_spec(1)] + [_full_spec(sh) for sh in wshapes])
    out_specs = [_row_spec(w) for w in out_widths]
    out_shape = [jax.ShapeDtypeStruct((nrows, w), jnp.float32)
                 for w in out_widths]
    return pl.pallas_call(
        body,
        grid=(nrows // BL,),
        in_specs=in_specs,
        out_specs=out_specs,
        out_shape=out_shape,
    )


_WSH_L = [(D, D), (1, D), (D, D), (1, D),
          (D, D), (1, D), (D, D), (1, D),
          (D, D), (D, D), (D, D), (1, D)]
_WSH_C = [(D, D), (1, D), (D, D), (1, D),
          (D, D), (D, D), (1, D)]

_l_step = _make_step(L, _l_step_body, [HD, HD], [D, HD, HD, D], _WSH_L)
_c_step = _make_step(C, _c_step_body, [HD, HD], [D, HD, HD, D], _WSH_C)


def kernel(l_size, c_size, l_edge_index, c_edge_index, l_emb, c_emb,
           l2c_W1, l2c_b1, l2c_W2, l2c_b2,
           c2l_W1, c2l_b1, c2l_W2, c2l_b2,
           l2l_W1, l2l_b1, l2l_W2, l2l_b2,
           c_upd_W, c_upd_b, l_upd_W, l_upd_b):
    f32 = jnp.float32
    i32 = jnp.int32
    pad = EP - E

    lsrc = jnp.concatenate([l_edge_index, jnp.zeros((pad,), i32)])
    csrc = jnp.concatenate([c_edge_index, jnp.zeros((pad,), i32)])
    ldst = jnp.concatenate([l_edge_index, jnp.full((pad,), L, i32)])
    cdst = jnp.concatenate([c_edge_index, jnp.full((pad,), C, i32)])

    n4 = 4 * MICRO
    n2 = 2 * MICRO
    zrows = jnp.zeros((LPT, HD), f32)
    z16 = jnp.zeros((HR, 16), f32)
    ident = jnp.arange(HR, dtype=i32)

    ldeg2, cdeg2 = _sc_degrees(
        ldst.reshape(-1, MACRO * MICRO), cdst.reshape(-1, MACRO * MICRO),
        z16, ident)
    ldeg = ldeg2.reshape(-1)[:L].reshape(L, 1)
    cdeg = cdeg2.reshape(-1)[:C].reshape(C, 1)

    Wca, Wcb = c_upd_W[:D], c_upd_W[D:]
    Wua, Wub, Wuc = l_upd_W[:D], l_upd_W[D:2 * D], l_upd_W[2 * D:]
    b1l, b2l = l2c_b1.reshape(1, D), l2c_b2.reshape(1, D)
    b1c, b2c = c2l_b1.reshape(1, D), c2l_b2.reshape(1, D)
    b1ll, b2ll = l2l_b1.reshape(1, D), l2l_b2.reshape(1, D)
    bc = c_upd_b.reshape(1, D)
    blr = l_upd_b.reshape(1, D)

    wl = (l2c_W1, b1l, l2c_W2, b2l, l2l_W1, b1ll, l2l_W2, b2ll,
          Wua, Wub, Wuc, blr)
    wc = (c2l_W1, b1c, c2l_W2, b2c, Wca, Wcb, bc)

    zal = jnp.zeros((LA, HD), f32)
    zac = jnp.zeros((CA, HD), f32)

    l_list = [l_emb]
    c_list = [c_emb]

    _, mllo, mlhi, plc = _l_step(l_emb, zal, zal, ldeg, *wl)
    _, mclo, mchi, pcc = _c_step(c_emb, zac, zac, cdeg, *wc)

    for _ in range(NIT):
        aggc_lo, aggc_hi = _sc_l2c(
            mllo, mlhi, lsrc.reshape(-1, n4), cdst.reshape(-1, n4), zrows)
        aggl_lo, aggl_hi = _sc_c2l(
            mclo, mchi, csrc.reshape(-1, n2), ldst.reshape(-1, n2), zrows)
        cnew, mclo, mchi, pcc = _c_step(
            pcc, aggc_lo, aggc_hi, cdeg, *wc)
        lnew, mllo, mlhi, plc = _l_step(
            plc, aggl_lo, aggl_hi, ldeg, *wl)
        c_list.append(cnew)
        l_list.append(lnew)

    return tuple(l_list) + tuple(c_list)


# speedup vs baseline: 1.0150x; 1.0150x over previous
import functools

import jax
import jax.numpy as jnp
from jax import lax
from jax.experimental import pallas as pl
from jax.experimental.pallas import tpu as pltpu
from jax.experimental.pallas import tpu_sc as plsc

D = 128
HD = 64
L = 20000
C = 10000
E = 600000
NIT = 4

NSUB = 16
MICRO = 128
MACRO = 4
EPT = 37888
EP = EPT * NSUB
NMAC = EPT // (MICRO * MACRO)
RPT = EPT // MICRO
NROW = EP // MICRO

CA = 10112
LA = 20096
CPT = CA // NSUB
LPT = LA // NSUB

BL = 1000


def _sc_phase(r0, srcN, tbl, dstN, acc, idxs, idxd, rows, gsem, ssem, isem,
              n, nmac):
    nchunk = n // MICRO
    pltpu.async_copy(srcN.at[r0], idxs.at[0], isem)
    pltpu.async_copy(dstN.at[r0], idxd.at[0], isem)

    def drain(sem, par):
        for j in range(nchunk):
            pltpu.make_async_copy(
                tbl.at[pl.ds(0, MICRO)],
                rows.at[par, pl.ds(j * MICRO, MICRO)], sem).wait()

    def body(m, carry):
        b = lax.rem(m, 2)
        p = lax.rem(m, 3)

        @pl.when(m >= 2)
        def _():
            drain(ssem, b)

        pltpu.make_async_copy(srcN.at[r0], idxs.at[p], isem).wait()
        pltpu.make_async_copy(dstN.at[r0], idxd.at[p], isem).wait()
        pltpu.async_copy(tbl.at[idxs.at[p]], rows.at[b], gsem)

        @pl.when(m + 1 < nmac)
        def _():
            pn = lax.rem(m + 1, 3)
            pltpu.async_copy(srcN.at[r0 + m + 1], idxs.at[pn], isem)
            pltpu.async_copy(dstN.at[r0 + m + 1], idxd.at[pn], isem)

        @pl.when(m >= 1)
        def _():
            pm = lax.rem(m + 2, 3)
            drain(gsem, 1 - b)
            pltpu.async_copy(rows.at[1 - b], acc.at[idxd.at[pm]], ssem,
                             add=True)

        return carry

    lax.fori_loop(0, nmac, body, 0)
    blast = (nmac - 1) % 2
    plast = (nmac - 1) % 3
    drain(gsem, blast)
    pltpu.async_copy(rows.at[blast], acc.at[idxd.at[plast]], ssem, add=True)
    drain(ssem, 0)
    drain(ssem, 1)


def _make_seg_sum(nacc, per_tile, macro):
    n = macro * MICRO
    nmac = EPT // n

    @functools.partial(
        pl.kernel,
        out_type=(
            jax.ShapeDtypeStruct((nacc, HD), jnp.float32),
            jax.ShapeDtypeStruct((nacc, HD), jnp.float32),
        ),
        mesh=plsc.VectorSubcoreMesh(core_axis_name="c", subcore_axis_name="s"),
        compiler_params=pltpu.CompilerParams(use_tc_tiling_on_sc=False),
        scratch_types=[
            pltpu.VMEM_SHARED((nacc, HD), jnp.float32),
            pltpu.VMEM((3, n), jnp.int32),
            pltpu.VMEM((3, n), jnp.int32),
            pltpu.VMEM((2, n, HD), jnp.float32),
            pltpu.SemaphoreType.DMA,
            pltpu.SemaphoreType.DMA,
            pltpu.SemaphoreType.DMA,
        ],
    )
    def seg_sum(tbl0, tbl1, srcN, dstN, zrows,
                out0, out1, acc, idxs, idxd, rows, gsem, ssem, isem):
        core = lax.axis_index("c")
        sub = lax.axis_index("s")
        pltpu.sync_copy(zrows.at[pl.ds(0, per_tile)],
                        acc.at[pl.ds(sub * per_tile, per_tile)])
        plsc.subcore_barrier()

        @pl.when(core == 0)
        def _():
            _sc_phase(sub * nmac, srcN, tbl0, dstN, acc, idxs, idxd, rows,
                      gsem, ssem, isem, n, nmac)

        @pl.when(core == 1)
        def _():
            _sc_phase(sub * nmac, srcN, tbl1, dstN, acc, idxs, idxd, rows,
                      gsem, ssem, isem, n, nmac)

        plsc.subcore_barrier()

        @pl.when(core == 0)
        def _():
            pltpu.sync_copy(acc.at[pl.ds(sub * per_tile, per_tile)],
                            out0.at[pl.ds(sub * per_tile, per_tile)])

        @pl.when(core == 1)
        def _():
            pltpu.sync_copy(acc.at[pl.ds(sub * per_tile, per_tile)],
                            out1.at[pl.ds(sub * per_tile, per_tile)])

    return seg_sum


_sc_l2c = _make_seg_sum(CA, CPT, 4)
_sc_c2l = _make_seg_sum(LA, LPT, 2)


@functools.partial(
    pl.kernel,
    out_type=(
        jax.ShapeDtypeStruct((LA, 16), jnp.float32),
        jax.ShapeDtypeStruct((CA, 16), jnp.float32),
    ),
    mesh=plsc.VectorSubcoreMesh(core_axis_name="c", subcore_axis_name="s"),
    compiler_params=pltpu.CompilerParams(use_tc_tiling_on_sc=False),
    scratch_types=[
        pltpu.VMEM_SHARED((LA, 16), jnp.float32),
        pltpu.VMEM((3, MACRO * MICRO), jnp.int32),
        pltpu.VMEM((MACRO * MICRO, 16), jnp.float32),
        pltpu.SemaphoreType.DMA,
        pltpu.SemaphoreType.DMA,
    ],
)
def _sc_degrees(ldst, cdst, z16, ones16, ldeg, cdeg, dacc, idxd, ones_v,
                ssem, isem):
    core = lax.axis_index("c")
    sub = lax.axis_index("s")
    nd = MACRO * MICRO
    pltpu.sync_copy(ones16, ones_v)

    @pl.when(core == 0)
    def _():
        pltpu.sync_copy(z16, dacc.at[pl.ds(sub * LPT, LPT)])

    @pl.when(core == 1)
    def _():
        pltpu.sync_copy(z16.at[pl.ds(0, CPT)], dacc.at[pl.ds(sub * CPT, CPT)])

    plsc.subcore_barrier()

    def deg_phase(dstN):
        r0 = sub * NMAC
        pltpu.async_copy(dstN.at[r0], idxd.at[0], isem)

        def body(m, carry):
            p = lax.rem(m, 3)

            @pl.when(m >= 2)
            def _():
                for j in range(MACRO):
                    pltpu.make_async_copy(
                        z16.at[pl.ds(0, MICRO)],
                        ones_v.at[pl.ds(j * MICRO, MICRO)], ssem).wait()

            pltpu.make_async_copy(dstN.at[r0], idxd.at[p], isem).wait()

            @pl.when(m + 1 < NMAC)
            def _():
                pn = lax.rem(m + 1, 3)
                pltpu.async_copy(dstN.at[r0 + m + 1], idxd.at[pn], isem)

            pltpu.async_copy(ones_v, dacc.at[idxd.at[p]], ssem, add=True)

            return carry

        lax.fori_loop(0, NMAC, body, 0)
        for _ in range(2):
            for j in range(MACRO):
                pltpu.make_async_copy(
                    z16.at[pl.ds(0, MICRO)],
                    ones_v.at[pl.ds(j * MICRO, MICRO)], ssem).wait()

    @pl.when(core == 0)
    def _():
        deg_phase(ldst)

    @pl.when(core == 1)
    def _():
        deg_phase(cdst)

    plsc.subcore_barrier()

    @pl.when(core == 0)
    def _():
        pltpu.sync_copy(dacc.at[pl.ds(sub * LPT, LPT)],
                        ldeg.at[pl.ds(sub * LPT, LPT)])

    @pl.when(core == 1)
    def _():
        pltpu.sync_copy(dacc.at[pl.ds(sub * CPT, CPT)],
                        cdeg.at[pl.ds(sub * CPT, CPT)])


def _dot(a, b):
    return jnp.dot(a, b, preferred_element_type=jnp.float32)


def _l_step_body(pin, agg_lo, agg_hi, deg,
                 W1, b1, W2, b2, lW1, lb1, lW2, lb2, Wua, Wub, Wuc, bl,
                 lnew_ref, mlo_ref, mhi_ref, pout_ref):
    a = jnp.concatenate([agg_lo[...], agg_hi[...]], axis=1)
    d = deg[...][:, 0:1]
    s = jnp.where(d > 0, lax.rsqrt(d), 1.0)
    lnew = pin[...] + _dot(a * s, Wub[...])
    lnew_ref[...] = lnew
    h = jnp.maximum(_dot(lnew, W1[...]) + b1[...], 0.0)
    y = _dot(h, W2[...]) + b2[...]
    mlv = y * s
    mlo_ref[...] = mlv[:, :HD]
    mhi_ref[...] = mlv[:, HD:]
    h2 = jnp.maximum(_dot(lnew, lW1[...]) + lb1[...], 0.0)
    y2 = _dot(h2, lW2[...]) + lb2[...]
    z = _dot(y2, Wuc[...])
    r = lax.broadcasted_iota(jnp.int32, (BL, BL), 0)
    c = lax.broadcasted_iota(jnp.int32, (BL, BL), 1)
    sw = (c == (r ^ 1)).astype(jnp.float32)
    pout_ref[...] = _dot(lnew, Wua[...]) + _dot(sw, z) + bl[...]


def _c_step_body(pin, agg_lo, agg_hi, deg,
                 W1, b1, W2, b2, Wca, Wcb, bc,
                 cnew_ref, mlo_ref, mhi_ref, pout_ref):
    a = jnp.concatenate([agg_lo[...], agg_hi[...]], axis=1)
    d = deg[...][:, 0:1]
    s = jnp.where(d > 0, lax.rsqrt(d), 1.0)
    cnew = pin[...] + _dot(a * s, Wcb[...])
    cnew_ref[...] = cnew
    h = jnp.maximum(_dot(cnew, W1[...]) + b1[...], 0.0)
    y = _dot(h, W2[...]) + b2[...]
    mcv = y * s
    mlo_ref[...] = mcv[:, :HD]
    mhi_ref[...] = mcv[:, HD:]
    pout_ref[...] = _dot(cnew, Wca[...]) + bc[...]


def _row_spec(cols):
    return pl.BlockSpec((BL, cols), lambda i: (i, 0))


def _full_spec(shape):
    n = len(shape)
    return pl.BlockSpec(shape, lambda i: (0,) * n)


def _make_step(nrows, body, agg_widths, out_widths, wshapes):
    in_specs = ([_row_spec(D)] + [_row_spec(w) for w in agg_widths]
                + [_row_spec(16)] + [_full_spec(sh) for sh in wshapes])
    out_specs = [_row_spec(w) for w in out_widths]
    out_shape = [jax.ShapeDtypeStruct((nrows, w), jnp.float32)
                 for w in out_widths]
    return pl.pallas_call(
        body,
        grid=(nrows // BL,),
        in_specs=in_specs,
        out_specs=out_specs,
        out_shape=out_shape,
    )


_WSH_L = [(D, D), (1, D), (D, D), (1, D),
          (D, D), (1, D), (D, D), (1, D),
          (D, D), (D, D), (D, D), (1, D)]
_WSH_C = [(D, D), (1, D), (D, D), (1, D),
          (D, D), (D, D), (1, D)]

_l_step = _make_step(L, _l_step_body, [HD, HD], [D, HD, HD, D], _WSH_L)
_c_step = _make_step(C, _c_step_body, [HD, HD], [D, HD, HD, D], _WSH_C)


def kernel(l_size, c_size, l_edge_index, c_edge_index, l_emb, c_emb,
           l2c_W1, l2c_b1, l2c_W2, l2c_b2,
           c2l_W1, c2l_b1, c2l_W2, c2l_b2,
           l2l_W1, l2l_b1, l2l_W2, l2l_b2,
           c_upd_W, c_upd_b, l_upd_W, l_upd_b):
    f32 = jnp.float32
    i32 = jnp.int32
    pad = EP - E

    lsrc = jnp.concatenate([l_edge_index, jnp.zeros((pad,), i32)])
    csrc = jnp.concatenate([c_edge_index, jnp.zeros((pad,), i32)])
    ldst = jnp.concatenate([l_edge_index, jnp.full((pad,), L, i32)])
    cdst = jnp.concatenate([c_edge_index, jnp.full((pad,), C, i32)])

    n4 = 4 * MICRO
    n2 = 2 * MICRO
    zrows = jnp.zeros((LPT, HD), f32)
    z16 = jnp.zeros((LPT, 16), f32)
    ones16 = jnp.ones((MACRO * MICRO, 16), f32)

    ldeg, cdeg = _sc_degrees(
        ldst.reshape(-1, MACRO * MICRO), cdst.reshape(-1, MACRO * MICRO),
        z16, ones16)

    Wca, Wcb = c_upd_W[:D], c_upd_W[D:]
    Wua, Wub, Wuc = l_upd_W[:D], l_upd_W[D:2 * D], l_upd_W[2 * D:]
    b1l, b2l = l2c_b1.reshape(1, D), l2c_b2.reshape(1, D)
    b1c, b2c = c2l_b1.reshape(1, D), c2l_b2.reshape(1, D)
    b1ll, b2ll = l2l_b1.reshape(1, D), l2l_b2.reshape(1, D)
    bc = c_upd_b.reshape(1, D)
    blr = l_upd_b.reshape(1, D)

    wl = (l2c_W1, b1l, l2c_W2, b2l, l2l_W1, b1ll, l2l_W2, b2ll,
          Wua, Wub, Wuc, blr)
    wc = (c2l_W1, b1c, c2l_W2, b2c, Wca, Wcb, bc)

    zal = jnp.zeros((LA, HD), f32)
    zac = jnp.zeros((CA, HD), f32)

    l_list = [l_emb]
    c_list = [c_emb]

    _, mllo, mlhi, plc = _l_step(l_emb, zal, zal, ldeg, *wl)
    _, mclo, mchi, pcc = _c_step(c_emb, zac, zac, cdeg, *wc)

    for _ in range(NIT):
        aggc_lo, aggc_hi = _sc_l2c(
            mllo, mlhi, lsrc.reshape(-1, n4), cdst.reshape(-1, n4), zrows)
        aggl_lo, aggl_hi = _sc_c2l(
            mclo, mchi, csrc.reshape(-1, n2), ldst.reshape(-1, n2), zrows)
        cnew, mclo, mchi, pcc = _c_step(
            pcc, aggc_lo, aggc_hi, cdeg, *wc)
        lnew, mllo, mlhi, plc = _l_step(
            plc, aggl_lo, aggl_hi, ldeg, *wl)
        c_list.append(cnew)
        l_list.append(lnew)

    return tuple(l_list) + tuple(c_list)
